# baseline (speedup 1.0000x reference)
import jax
import jax.numpy as jnp
from jax import lax
from jax.experimental import pallas as pl
from jax.experimental.pallas import tpu as pltpu
from jax.experimental.pallas import tpu_sc as plsc

N = 100000
D = 128
NW = 32
C = 448
NCH = 7
RPW = NCH * C
NPAD = NW * RPW
NPAIR = (NCH - 1) // 2
TAIL = N - (NW - 1) * RPW - (NCH - 1) * C


def _gather_body(x_hbm, idx_hbm, out_hbm, idx_v, buf0, buf1, g0, g1):
    wid = lax.axis_index("s") * 2 + lax.axis_index("c")
    base = pl.multiple_of(wid * RPW, RPW)
    last = wid == NW - 1
    pltpu.sync_copy(idx_hbm.at[pl.ds(base, RPW)], idx_v)

    def gather(k, buf, sem):
        off = pl.multiple_of(k * C, C)
        return pltpu.async_copy(x_hbm.at[idx_v.at[pl.ds(off, C)]], buf, sem)

    def gwait(k, buf, sem):
        off = pl.multiple_of(k * C, C)
        pltpu.make_async_copy(x_hbm.at[idx_v.at[pl.ds(off, C)]], buf, sem).wait()

    def store(k, buf):
        pltpu.sync_copy(buf, out_hbm.at[pl.ds(base + k * C, C)])

    gather(0, buf0, g0)

    def pair(i, carry):
        k0 = 2 * i
        gather(k0 + 1, buf1, g1)
        gwait(k0, buf0, g0)
        store(k0, buf0)
        gather(k0 + 2, buf0, g0)
        gwait(k0 + 1, buf1, g1)
        store(k0 + 1, buf1)
        return carry

    lax.fori_loop(0, NPAIR, pair, 0)

    gwait(NCH - 1, buf0, g0)

    @pl.when(jnp.logical_not(last))
    def _():
        store(NCH - 1, buf0)

    @pl.when(last)
    def _():
        pltpu.sync_copy(
            buf0.at[pl.ds(0, TAIL)],
            out_hbm.at[pl.ds(base + (NCH - 1) * C, TAIL)],
        )


@jax.jit
def _gather(x, idx):
    mesh = plsc.VectorSubcoreMesh(core_axis_name="c", subcore_axis_name="s")
    f = pl.kernel(
        _gather_body,
        out_type=jax.ShapeDtypeStruct((N, D), jnp.float32),
        mesh=mesh,
        scratch_types=[
            pltpu.VMEM((RPW,), jnp.int32),
            pltpu.VMEM((C, D), jnp.float32),
            pltpu.VMEM((C, D), jnp.float32),
            pltpu.SemaphoreType.DMA,
            pltpu.SemaphoreType.DMA,
        ],
    )
    return f(x, idx)


def kernel(x, cell_type_indices, permutations):
    idx = permutations.reshape(-1).astype(jnp.int32)
    idx = jnp.concatenate([idx, jnp.zeros((NPAD - N,), jnp.int32)])
    return _gather(x, idx)
